# baseline (speedup 1.0000x reference)
import functools

import jax
import jax.numpy as jnp
from jax import lax
from jax.experimental import pallas as pl
from jax.experimental.pallas import tpu as pltpu
from jax.experimental.pallas import tpu_sc as plsc

DIM = 64
PAD_DIM = 128
SEQ = 50
BATCH = 16384
NUM_WORKERS = 32
B_PER_W = BATCH // NUM_WORKERS
QB = 128
NQ = B_PER_W // QB


def _sc_gather(table_p, idx_t):
    mesh = plsc.VectorSubcoreMesh(core_axis_name="c", subcore_axis_name="s")

    @functools.partial(
        pl.kernel,
        mesh=mesh,
        compiler_params=pltpu.CompilerParams(
            use_tc_tiling_on_sc=True, needs_layout_passes=False),
        out_type=jax.ShapeDtypeStruct((SEQ, DIM, BATCH), jnp.float32),
        scratch_types=[
            pltpu.VMEM((NQ, SEQ, QB), jnp.int32),
            pltpu.VMEM((2, QB, PAD_DIM), jnp.float32),
            pltpu.VMEM((2, DIM, QB), jnp.float32),
            pltpu.SemaphoreType.DMA,
            pltpu.SemaphoreType.DMA,
            pltpu.SemaphoreType.DMA,
            pltpu.SemaphoreType.DMA,
        ],
    )
    def k(table_hbm, idx_hbm, out_hbm, idx_v, a_v, t_v,
          sg0, sg1, so0, so1):
        semg = (sg0, sg1)
        semo = (so0, so1)
        wid = lax.axis_index("s") * 2 + lax.axis_index("c")
        b0 = wid * B_PER_W

        for q in range(NQ):
            pltpu.sync_copy(
                idx_hbm.at[:, pl.ds(b0 + q * QB, QB)], idx_v.at[q])

        def start_gather(l, q, n):
            src = table_hbm.at[idx_v.at[q, l]]
            pltpu.async_copy(src, a_v.at[n], semg[n])

        def wait_gather(n):
            pltpu.make_async_copy(
                table_hbm.at[pl.ds(0, QB)], a_v.at[n], semg[n]).wait()

        def start_write(l, q, n):
            pltpu.async_copy(
                t_v.at[n], out_hbm.at[l, :, pl.ds(b0 + q * QB, QB)],
                semo[n])

        def wait_write(n):
            pltpu.make_async_copy(
                t_v.at[n], out_hbm.at[0, :, pl.ds(b0 + n * QB, QB)],
                semo[n]).wait()

        def transpose(n):
            iota = lax.iota(jnp.int32, 16)

            def body(g, carry):
                rows = iota + g * 16
                for d in range(DIM):
                    v = plsc.load_gather(
                        a_v.at[n], [rows, jnp.full((16,), d, jnp.int32)])
                    t_v[n, d, pl.ds(g * 16, 16)] = v
                return carry

            lax.fori_loop(0, QB // 16, body, 0)

        def nxt(l, q):
            if q < 2:
                return l, q + 2
            return l + 1, q - 2

        start_gather(0, 0, 0)
        start_gather(0, 1, 1)
        for q in range(NQ):
            n = q % 2
            wait_gather(n)
            if q >= 2:
                wait_write(n)
            transpose(n)
            start_write(0, q, n)
            nl, nq = nxt(0, q)
            start_gather(nl, nq, n)

        def step(l, carry):
            for q in range(NQ):
                n = q % 2
                wait_gather(n)
                wait_write(n)
                transpose(n)
                start_write(l, q, n)
                nl, nq = nxt(l, q)
                start_gather(nl, nq, n)
            return carry

        lax.fori_loop(1, SEQ - 1, step, 0)

        for q in range(NQ):
            n = q % 2
            wait_gather(n)
            wait_write(n)
            transpose(n)
            start_write(SEQ - 1, q, n)
            if q < 2:
                start_gather(SEQ - 1, q + 2, n)
        for n in range(2):
            wait_write(n)

    return k(table_p, idx_t)


def kernel(bos_tensor, table):
    idx_t = jnp.transpose(bos_tensor).astype(jnp.int32)
    table_p = jnp.pad(table, ((0, 0), (0, PAD_DIM - DIM)))
    out_t = _sc_gather(table_p, idx_t)
    return jnp.transpose(out_t, (2, 0, 1))


# speedup vs baseline: 1.9652x; 1.9652x over previous
import functools

import jax
import jax.numpy as jnp
from jax import lax
from jax.experimental import pallas as pl
from jax.experimental.pallas import tpu as pltpu
from jax.experimental.pallas import tpu_sc as plsc

DIM = 64
PAD_DIM = 128
SEQ = 50
BATCH = 16384
NUM_WORKERS = 32
B_PER_W = BATCH // NUM_WORKERS
QB = 128
NQ = B_PER_W // QB


def _sc_gather(table_p, idx_t):
    mesh = plsc.VectorSubcoreMesh(core_axis_name="c", subcore_axis_name="s")

    @functools.partial(
        pl.kernel,
        mesh=mesh,
        compiler_params=pltpu.CompilerParams(
            use_tc_tiling_on_sc=True, needs_layout_passes=False),
        out_type=jax.ShapeDtypeStruct((SEQ, DIM, BATCH), jnp.float32),
        scratch_types=[
            pltpu.VMEM((NQ, SEQ, QB), jnp.int32),
            pltpu.VMEM((2, QB, PAD_DIM), jnp.float32),
            pltpu.VMEM((2, DIM, QB), jnp.float32),
            pltpu.SemaphoreType.DMA,
            pltpu.SemaphoreType.DMA,
            pltpu.SemaphoreType.DMA,
            pltpu.SemaphoreType.DMA,
        ],
    )
    def k(table_hbm, idx_hbm, out_hbm, idx_v, a_v, t_v,
          sg0, sg1, so0, so1):
        semg = (sg0, sg1)
        semo = (so0, so1)
        wid = lax.axis_index("s") * 2 + lax.axis_index("c")
        b0 = wid * B_PER_W

        for q in range(NQ):
            pltpu.sync_copy(
                idx_hbm.at[:, pl.ds(b0 + q * QB, QB)], idx_v.at[q])

        def start_gather(l, q, n):
            src = table_hbm.at[idx_v.at[q, l]]
            pltpu.async_copy(src, a_v.at[n], semg[n])

        def wait_gather(n):
            pltpu.make_async_copy(
                table_hbm.at[pl.ds(0, QB)], a_v.at[n], semg[n]).wait()

        def start_write(l, q, n):
            pltpu.async_copy(
                t_v.at[n], out_hbm.at[l, :, pl.ds(b0 + q * QB, QB)],
                semo[n])

        def wait_write(n):
            pltpu.make_async_copy(
                t_v.at[n], out_hbm.at[0, :, pl.ds(b0 + n * QB, QB)],
                semo[n]).wait()

        def transpose(n):
            iota = lax.iota(jnp.int32, 16)

            def body(g, carry):
                rows = iota + g * 16
                for dblk in range(DIM // 16):
                    for kk in range(16):
                        diag = (iota + kk) & 15
                        cols = diag + dblk * 16
                        v = plsc.load_gather(a_v.at[n], [rows, cols])
                        plsc.store_scatter(t_v.at[n], [cols, rows], v)
                return carry

            lax.fori_loop(0, QB // 16, body, 0)

        def nxt(l, q):
            if q < 2:
                return l, q + 2
            return l + 1, q - 2

        start_gather(0, 0, 0)
        start_gather(0, 1, 1)
        for q in range(NQ):
            n = q % 2
            wait_gather(n)
            if q >= 2:
                wait_write(n)
            transpose(n)
            start_write(0, q, n)
            nl, nq = nxt(0, q)
            start_gather(nl, nq, n)

        def step(l, carry):
            for q in range(NQ):
                n = q % 2
                wait_gather(n)
                wait_write(n)
                transpose(n)
                start_write(l, q, n)
                nl, nq = nxt(l, q)
                start_gather(nl, nq, n)
            return carry

        lax.fori_loop(1, SEQ - 1, step, 0)

        for q in range(NQ):
            n = q % 2
            wait_gather(n)
            wait_write(n)
            transpose(n)
            start_write(SEQ - 1, q, n)
            if q < 2:
                start_gather(SEQ - 1, q + 2, n)
        for n in range(2):
            wait_write(n)

    return k(table_p, idx_t)


def kernel(bos_tensor, table):
    idx_t = jnp.transpose(bos_tensor).astype(jnp.int32)
    table_p = jnp.pad(table, ((0, 0), (0, PAD_DIM - DIM)))
    out_t = _sc_gather(table_p, idx_t)
    return jnp.transpose(out_t, (2, 0, 1))


# speedup vs baseline: 2.7697x; 1.4094x over previous
import functools

import jax
import jax.numpy as jnp
from jax import lax
from jax.experimental import pallas as pl
from jax.experimental.pallas import tpu as pltpu
from jax.experimental.pallas import tpu_sc as plsc

DIM = 64
PAD_DIM = 128
SEQ = 50
BATCH = 16384
NUM_WORKERS = 32
B_PER_W = BATCH // NUM_WORKERS
QB = 128
NQ = B_PER_W // QB


def _sc_gather(table_p, idx_t):
    mesh = plsc.VectorSubcoreMesh(core_axis_name="c", subcore_axis_name="s")

    @functools.partial(
        pl.kernel,
        mesh=mesh,
        compiler_params=pltpu.CompilerParams(
            use_tc_tiling_on_sc=True, needs_layout_passes=False),
        out_type=jax.ShapeDtypeStruct((SEQ, DIM, BATCH), jnp.float32),
        scratch_types=[
            pltpu.VMEM((NQ, SEQ, QB), jnp.int32),
            pltpu.VMEM((2, QB, PAD_DIM), jnp.float32),
            pltpu.VMEM((2, DIM, QB), jnp.float32),
            pltpu.SemaphoreType.DMA,
            pltpu.SemaphoreType.DMA,
            pltpu.SemaphoreType.DMA,
            pltpu.SemaphoreType.DMA,
        ],
    )
    def k(table_hbm, idx_hbm, out_hbm, idx_v, a_v, t_v,
          sg0, sg1, so0, so1):
        semg = (sg0, sg1)
        semo = (so0, so1)
        wid = lax.axis_index("s") * 2 + lax.axis_index("c")
        b0 = wid * B_PER_W

        for q in range(NQ):
            pltpu.sync_copy(
                idx_hbm.at[:, pl.ds(b0 + q * QB, QB)], idx_v.at[q])

        def start_gather(l, q, n):
            src = table_hbm.at[idx_v.at[q, l]]
            pltpu.async_copy(src, a_v.at[n], semg[n])

        def wait_gather(n):
            pltpu.make_async_copy(
                table_hbm.at[pl.ds(0, QB)], a_v.at[n], semg[n]).wait()

        def start_write(l, q, n):
            pltpu.async_copy(
                t_v.at[n], out_hbm.at[l, :, pl.ds(b0 + q * QB, QB)],
                semo[n])

        def wait_write(n):
            pltpu.make_async_copy(
                t_v.at[n], out_hbm.at[0, :, pl.ds(b0 + n * QB, QB)],
                semo[n]).wait()

        def transpose(n):
            iota = lax.iota(jnp.int32, 16)

            def body(g2, carry):
                for gg in range(2):
                    gbase = (g2 * 2 + gg) * 16
                    for kk in range(16):
                        rows = ((iota + kk) & 15) + gbase
                        for dblk in range(DIM // 16):
                            cols = iota + dblk * 16
                            v = plsc.load_gather(a_v.at[n], [rows, cols])
                            plsc.store_scatter(t_v.at[n], [cols, rows], v)
                return carry

            lax.fori_loop(0, QB // 32, body, 0)

        def nxt(l, q):
            if q < 2:
                return l, q + 2
            return l + 1, q - 2

        start_gather(0, 0, 0)
        start_gather(0, 1, 1)
        for q in range(NQ):
            n = q % 2
            wait_gather(n)
            if q >= 2:
                wait_write(n)
            transpose(n)
            start_write(0, q, n)
            nl, nq = nxt(0, q)
            start_gather(nl, nq, n)

        def step(l, carry):
            for q in range(NQ):
                n = q % 2
                wait_gather(n)
                wait_write(n)
                transpose(n)
                start_write(l, q, n)
                nl, nq = nxt(l, q)
                start_gather(nl, nq, n)
            return carry

        lax.fori_loop(1, SEQ - 1, step, 0)

        for q in range(NQ):
            n = q % 2
            wait_gather(n)
            wait_write(n)
            transpose(n)
            start_write(SEQ - 1, q, n)
            if q < 2:
                start_gather(SEQ - 1, q + 2, n)
        for n in range(2):
            wait_write(n)

    return k(table_p, idx_t)


def kernel(bos_tensor, table):
    idx_t = jnp.transpose(bos_tensor).astype(jnp.int32)
    table_p = jnp.pad(table, ((0, 0), (0, PAD_DIM - DIM)))
    out_t = _sc_gather(table_p, idx_t)
    return jnp.transpose(out_t, (2, 0, 1))


# speedup vs baseline: 2.7990x; 1.0106x over previous
import functools

import jax
import jax.numpy as jnp
from jax import lax
from jax.experimental import pallas as pl
from jax.experimental.pallas import tpu as pltpu
from jax.experimental.pallas import tpu_sc as plsc

DIM = 64
PAD_DIM = 128
SEQ = 50
BATCH = 16384
NUM_WORKERS = 32
B_PER_W = BATCH // NUM_WORKERS
QB = 128
NQ = B_PER_W // QB


def _sc_gather(table_p, idx_t):
    mesh = plsc.VectorSubcoreMesh(core_axis_name="c", subcore_axis_name="s")

    @functools.partial(
        pl.kernel,
        mesh=mesh,
        compiler_params=pltpu.CompilerParams(
            use_tc_tiling_on_sc=True, needs_layout_passes=False),
        out_type=jax.ShapeDtypeStruct((SEQ, DIM, BATCH), jnp.float32),
        scratch_types=[
            pltpu.VMEM((NQ, SEQ, QB), jnp.int32),
            pltpu.VMEM((NQ, QB, PAD_DIM), jnp.float32),
            pltpu.VMEM((2, DIM, QB), jnp.float32),
            pltpu.SemaphoreType.DMA,
            pltpu.SemaphoreType.DMA,
            pltpu.SemaphoreType.DMA,
            pltpu.SemaphoreType.DMA,
            pltpu.SemaphoreType.DMA,
            pltpu.SemaphoreType.DMA,
        ],
    )
    def k(table_hbm, idx_hbm, out_hbm, idx_v, a_v, t_v,
          sg0, sg1, sg2, sg3, so0, so1):
        semg = (sg0, sg1, sg2, sg3)
        semo = (so0, so1)
        wid = lax.axis_index("s") * 2 + lax.axis_index("c")
        b0 = wid * B_PER_W

        for q in range(NQ):
            pltpu.sync_copy(
                idx_hbm.at[:, pl.ds(b0 + q * QB, QB)], idx_v.at[q])

        def start_gather(l, q):
            src = table_hbm.at[idx_v.at[q, l]]
            pltpu.async_copy(src, a_v.at[q], semg[q])

        def wait_gather(q):
            pltpu.make_async_copy(
                table_hbm.at[pl.ds(0, QB)], a_v.at[q], semg[q]).wait()

        def start_write(l, q, m):
            pltpu.async_copy(
                t_v.at[m], out_hbm.at[l, :, pl.ds(b0 + q * QB, QB)],
                semo[m])

        def wait_write(m):
            pltpu.make_async_copy(
                t_v.at[m], out_hbm.at[0, :, pl.ds(b0 + m * QB, QB)],
                semo[m]).wait()

        def transpose(q, m):
            iota = lax.iota(jnp.int32, 16)

            def body(g2, carry):
                for gg in range(2):
                    gbase = (g2 * 2 + gg) * 16
                    for kk in range(16):
                        rows = (iota ^ kk) + gbase
                        for dblk in range(DIM // 16):
                            cols = iota + dblk * 16
                            v = plsc.load_gather(a_v.at[q], [rows, cols])
                            plsc.store_scatter(t_v.at[m], [cols, rows], v)
                return carry

            lax.fori_loop(0, QB // 32, body, 0)

        for q in range(NQ):
            start_gather(0, q)
        for q in range(NQ):
            m = q % 2
            wait_gather(q)
            if q >= 2:
                wait_write(m)
            transpose(q, m)
            start_write(0, q, m)
            start_gather(1, q)

        def step(l, carry):
            for q in range(NQ):
                m = q % 2
                wait_gather(q)
                wait_write(m)
                transpose(q, m)
                start_write(l, q, m)
                start_gather(l + 1, q)
            return carry

        lax.fori_loop(1, SEQ - 1, step, 0)

        for q in range(NQ):
            m = q % 2
            wait_gather(q)
            wait_write(m)
            transpose(q, m)
            start_write(SEQ - 1, q, m)
        for m in range(2):
            wait_write(m)

    return k(table_p, idx_t)


def kernel(bos_tensor, table):
    idx_t = jnp.transpose(bos_tensor).astype(jnp.int32)
    table_p = jnp.pad(table, ((0, 0), (0, PAD_DIM - DIM)))
    out_t = _sc_gather(table_p, idx_t)
    return jnp.transpose(out_t, (2, 0, 1))
